# baseline (speedup 1.0000x reference)
import functools

import jax
import jax.numpy as jnp
from jax.experimental import pallas as pl
from jax.experimental.pallas import tpu as pltpu

E = 8
TOPK = 2
D = 768
FF = 2048
R = 64
SCALING = 16.0 / 64.0

FBLK = 128
NFF = FF // FBLK
NTOK = 2048


def _ffn_kernel(x_ref, Wc_ref, bc_ref, wealth_ref,
                Wg_ref, Wu_ref, Wd_ref,
                Agu_ref, Bg_ref, Bu_ref, Ad_ref, Bd2_ref,
                out_ref, P_s, W8_s, PD_s, acc_s):
    j = pl.program_id(0)
    xb = x_ref[...]

    @pl.when(j == 0)
    def _():
        logits = jnp.dot(xb, Wc_ref[...].T, preferred_element_type=jnp.float32)
        bids = jax.nn.sigmoid(logits + bc_ref[...]) * wealth_ref[...]
        iota = jax.lax.broadcasted_iota(jnp.int32, (NTOK, E), 1)
        m1 = jnp.max(bids, axis=-1, keepdims=True)
        i1 = jnp.min(jnp.where(bids == m1, iota, E), axis=-1, keepdims=True)
        oh1 = iota == i1
        masked = jnp.where(oh1, -jnp.inf, bids)
        m2 = jnp.max(masked, axis=-1, keepdims=True)
        i2 = jnp.min(jnp.where(masked == m2, iota, E), axis=-1, keepdims=True)
        oh2 = iota == i2
        w1 = jax.nn.sigmoid(m1 - m2)
        W8_s[...] = jnp.where(oh1, w1, 0.0) + jnp.where(oh2, 1.0 - w1, 0.0)
        P_s[...] = jnp.dot(xb, Agu_ref[...],
                           preferred_element_type=jnp.float32).astype(jnp.bfloat16)

    G0 = jnp.dot(xb, Wg_ref[...], preferred_element_type=jnp.float32)
    U0 = jnp.dot(xb, Wu_ref[...], preferred_element_type=jnp.float32)

    W8 = W8_s[...]
    H = jnp.zeros((NTOK, FBLK), dtype=jnp.float32)
    pds = []
    for e in range(E):
        dg = jnp.dot(P_s[:, e * R:(e + 1) * R], Bg_ref[e],
                     preferred_element_type=jnp.float32) * SCALING
        du = jnp.dot(P_s[:, E * R + e * R:E * R + (e + 1) * R], Bu_ref[e],
                     preferred_element_type=jnp.float32) * SCALING
        h = jax.nn.silu(G0 + dg) * (U0 + du)
        hw = W8[:, e:e + 1] * h
        H = H + hw
        pds.append(jnp.dot(hw, Ad_ref[e], preferred_element_type=jnp.float32))

    PD = jnp.concatenate(pds, axis=1)
    y = jnp.dot(H, Wd_ref[...], preferred_element_type=jnp.float32)

    @pl.when(j == 0)
    def _():
        PD_s[...] = PD
        acc_s[...] = y

    @pl.when(j > 0)
    def _():
        PD_s[...] += PD
        acc_s[...] += y

    @pl.when(j == NFF - 1)
    def _():
        dlora = jnp.dot(PD_s[...], Bd2_ref[...],
                        preferred_element_type=jnp.float32)
        out_ref[...] = acc_s[...] + dlora * SCALING


@functools.partial(jax.jit, static_argnames=("interpret",))
def _run(x2d, Wc, bc, wealth, W_gate, W_up, W_down,
         A_gate, B_gate, A_up, B_up, A_down, B_down, interpret=False):
    full = lambda *shape: pl.BlockSpec(shape, lambda j: (0,) * len(shape))
    return pl.pallas_call(
        _ffn_kernel,
        grid=(NFF,),
        in_specs=[
            full(NTOK, D),
            full(E, D),
            full(1, E),
            full(1, E),
            pl.BlockSpec((D, FBLK), lambda j: (0, j)),
            pl.BlockSpec((D, FBLK), lambda j: (0, j)),
            pl.BlockSpec((FBLK, D), lambda j: (j, 0)),
            full(D, 2 * E * R),
            pl.BlockSpec((E, R, FBLK), lambda j: (0, 0, j)),
            pl.BlockSpec((E, R, FBLK), lambda j: (0, 0, j)),
            pl.BlockSpec((E, FBLK, R), lambda j: (0, j, 0)),
            full(E * R, D),
        ],
        out_specs=full(NTOK, D),
        out_shape=jax.ShapeDtypeStruct((NTOK, D), jnp.float32),
        scratch_shapes=[
            pltpu.VMEM((NTOK, 2 * E * R), jnp.bfloat16),
            pltpu.VMEM((NTOK, E), jnp.float32),
            pltpu.VMEM((NTOK, E * R), jnp.float32),
            pltpu.VMEM((NTOK, D), jnp.float32),
        ],
        interpret=interpret,
    )(x2d, Wc, bc.reshape(1, E), wealth.reshape(1, E), W_gate, W_up, W_down,
      jnp.concatenate([A_gate.transpose(1, 0, 2).reshape(D, E * R),
                       A_up.transpose(1, 0, 2).reshape(D, E * R)],
                      axis=1),
      B_gate, B_up, A_down, B_down.reshape(E * R, D))


def kernel(x, Wc, bc, wealth, W_gate, W_up, W_down,
           A_gate, B_gate, A_up, B_up, A_down, B_down):
    B, S, _ = x.shape
    out = _run(x.reshape(B * S, D), Wc, bc, wealth, W_gate, W_up, W_down,
               A_gate, B_gate, A_up, B_up, A_down, B_down)
    return out.reshape(B, S, D)


# speedup vs baseline: 1.2863x; 1.2863x over previous
import functools

import jax
import jax.numpy as jnp
from jax.experimental import pallas as pl

E = 8
TOPK = 2
D = 768
FF = 2048
R = 64
SCALING = 16.0 / 64.0

TBLK = 256


def _ffn_kernel(x_ref, Wc_ref, bc_ref, wealth_ref,
                Wg_ref, Wu_ref, Wd_ref,
                Agu_ref, Bg_ref, Bu_ref, Ad_ref, Bd2_ref,
                out_ref):
    xb = x_ref[...]

    logits = jnp.dot(xb, Wc_ref[...].T, preferred_element_type=jnp.float32)
    bids = jax.nn.sigmoid(logits + bc_ref[...]) * wealth_ref[...]
    iota = jax.lax.broadcasted_iota(jnp.int32, (TBLK, E), 1)
    m1 = jnp.max(bids, axis=-1, keepdims=True)
    i1 = jnp.min(jnp.where(bids == m1, iota, E), axis=-1, keepdims=True)
    oh1 = iota == i1
    masked = jnp.where(oh1, -jnp.inf, bids)
    m2 = jnp.max(masked, axis=-1, keepdims=True)
    i2 = jnp.min(jnp.where(masked == m2, iota, E), axis=-1, keepdims=True)
    oh2 = iota == i2
    w1 = jax.nn.sigmoid(m1 - m2)
    W8 = jnp.where(oh1, w1, 0.0) + jnp.where(oh2, 1.0 - w1, 0.0)

    xb16 = xb.astype(jnp.bfloat16)
    G0 = jnp.dot(xb16, Wg_ref[...], preferred_element_type=jnp.float32)
    U0 = jnp.dot(xb16, Wu_ref[...], preferred_element_type=jnp.float32)

    P = jnp.dot(xb, Agu_ref[...], preferred_element_type=jnp.float32)

    H = jnp.zeros((TBLK, FF), dtype=jnp.float32)
    pds = []
    for e in range(E):
        dg = jnp.dot(P[:, e * R:(e + 1) * R], Bg_ref[e],
                     preferred_element_type=jnp.float32) * SCALING
        du = jnp.dot(P[:, E * R + e * R:E * R + (e + 1) * R], Bu_ref[e],
                     preferred_element_type=jnp.float32) * SCALING
        h = jax.nn.silu(G0 + dg) * (U0 + du)
        hw = W8[:, e:e + 1] * h
        H = H + hw
        pds.append(jnp.dot(hw, Ad_ref[e], preferred_element_type=jnp.float32))

    PD = jnp.concatenate(pds, axis=1)
    dlora = jnp.dot(PD, Bd2_ref[...], preferred_element_type=jnp.float32)

    out_ref[...] = (jnp.dot(H.astype(jnp.bfloat16), Wd_ref[...],
                            preferred_element_type=jnp.float32)
                    + dlora * SCALING)


@functools.partial(jax.jit, static_argnames=("interpret",))
def _run(x2d, Wc, bc, wealth, W_gate, W_up, W_down,
         A_gate, B_gate, A_up, B_up, A_down, B_down, interpret=False):
    S = x2d.shape[0]
    grid = (S // TBLK,)
    full = lambda *shape: pl.BlockSpec(shape, lambda i: (0,) * len(shape))
    return pl.pallas_call(
        _ffn_kernel,
        grid=grid,
        in_specs=[
            pl.BlockSpec((TBLK, D), lambda i: (i, 0)),
            full(E, D),
            full(1, E),
            full(1, E),
            full(D, FF),
            full(D, FF),
            full(FF, D),
            full(D, 2 * E * R),
            full(E, R, FF),
            full(E, R, FF),
            full(E, FF, R),
            full(E * R, D),
        ],
        out_specs=pl.BlockSpec((TBLK, D), lambda i: (i, 0)),
        out_shape=jax.ShapeDtypeStruct((S, D), jnp.float32),
        interpret=interpret,
    )(x2d, Wc, bc.reshape(1, E), wealth.reshape(1, E),
      W_gate.astype(jnp.bfloat16), W_up.astype(jnp.bfloat16),
      W_down.astype(jnp.bfloat16),
      jnp.concatenate([A_gate.transpose(1, 0, 2).reshape(D, E * R),
                       A_up.transpose(1, 0, 2).reshape(D, E * R)],
                      axis=1),
      B_gate, B_up, A_down, B_down.reshape(E * R, D))


def kernel(x, Wc, bc, wealth, W_gate, W_up, W_down,
           A_gate, B_gate, A_up, B_up, A_down, B_down):
    B, S, _ = x.shape
    out = _run(x.reshape(B * S, D), Wc, bc, wealth, W_gate, W_up, W_down,
               A_gate, B_gate, A_up, B_up, A_down, B_down)
    return out.reshape(B, S, D)


# speedup vs baseline: 1.3534x; 1.0522x over previous
import functools

import jax
import jax.numpy as jnp
from jax.experimental import pallas as pl

E = 8
TOPK = 2
D = 768
FF = 2048
R = 64
SCALING = 16.0 / 64.0

TBLK = 256


def _ffn_kernel(x_ref, Wc_ref, bc_ref, wealth_ref,
                Wg_ref, Wu_ref, Wd_ref,
                Agu_ref, Bg_ref, Bu_ref, Ad_ref, Bd2_ref,
                out_ref):
    xb = x_ref[...]

    logits = jnp.dot(xb, Wc_ref[...].T, preferred_element_type=jnp.float32)
    bids = jax.nn.sigmoid(logits + bc_ref[...]) * wealth_ref[...]
    iota = jax.lax.broadcasted_iota(jnp.int32, (TBLK, E), 1)
    m1 = jnp.max(bids, axis=-1, keepdims=True)
    i1 = jnp.min(jnp.where(bids == m1, iota, E), axis=-1, keepdims=True)
    oh1 = iota == i1
    masked = jnp.where(oh1, -jnp.inf, bids)
    m2 = jnp.max(masked, axis=-1, keepdims=True)
    i2 = jnp.min(jnp.where(masked == m2, iota, E), axis=-1, keepdims=True)
    oh2 = iota == i2
    w1 = jax.nn.sigmoid(m1 - m2)
    W8 = jnp.where(oh1, w1, 0.0) + jnp.where(oh2, 1.0 - w1, 0.0)

    G0 = jnp.dot(xb, Wg_ref[...], preferred_element_type=jnp.float32)
    U0 = jnp.dot(xb, Wu_ref[...], preferred_element_type=jnp.float32)

    P = jnp.dot(xb, Agu_ref[...], preferred_element_type=jnp.float32)

    H = jnp.zeros((TBLK, FF), dtype=jnp.float32)
    pds = []
    for e in range(E):
        dg = jnp.dot(P[:, e * R:(e + 1) * R], Bg_ref[e],
                     preferred_element_type=jnp.float32) * SCALING
        du = jnp.dot(P[:, E * R + e * R:E * R + (e + 1) * R], Bu_ref[e],
                     preferred_element_type=jnp.float32) * SCALING
        h = jax.nn.silu(G0 + dg) * (U0 + du)
        hw = W8[:, e:e + 1] * h
        H = H + hw
        pds.append(jnp.dot(hw, Ad_ref[e], preferred_element_type=jnp.float32))

    PD = jnp.concatenate(pds, axis=1)
    dlora = jnp.dot(PD, Bd2_ref[...], preferred_element_type=jnp.float32)

    out_ref[...] = (jnp.dot(H, Wd_ref[...], preferred_element_type=jnp.float32)
                    + dlora * SCALING)


@functools.partial(jax.jit, static_argnames=("interpret",))
def _run(x2d, Wc, bc, wealth, W_gate, W_up, W_down,
         A_gate, B_gate, A_up, B_up, A_down, B_down, interpret=False):
    S = x2d.shape[0]
    grid = (S // TBLK,)
    full = lambda *shape: pl.BlockSpec(shape, lambda i: (0,) * len(shape))
    return pl.pallas_call(
        _ffn_kernel,
        grid=grid,
        in_specs=[
            pl.BlockSpec((TBLK, D), lambda i: (i, 0)),
            full(E, D),
            full(1, E),
            full(1, E),
            full(D, FF),
            full(D, FF),
            full(FF, D),
            full(D, 2 * E * R),
            full(E, R, FF),
            full(E, R, FF),
            full(E, FF, R),
            full(E * R, D),
        ],
        out_specs=pl.BlockSpec((TBLK, D), lambda i: (i, 0)),
        out_shape=jax.ShapeDtypeStruct((S, D), jnp.float32),
        interpret=interpret,
    )(x2d, Wc, bc.reshape(1, E), wealth.reshape(1, E), W_gate, W_up, W_down,
      jnp.concatenate([A_gate.transpose(1, 0, 2).reshape(D, E * R),
                       A_up.transpose(1, 0, 2).reshape(D, E * R)],
                      axis=1),
      B_gate, B_up, A_down, B_down.reshape(E * R, D))


def kernel(x, Wc, bc, wealth, W_gate, W_up, W_down,
           A_gate, B_gate, A_up, B_up, A_down, B_down):
    B, S, _ = x.shape
    out = _run(x.reshape(B * S, D), Wc, bc, wealth, W_gate, W_up, W_down,
               A_gate, B_gate, A_up, B_up, A_down, B_down)
    return out.reshape(B, S, D)


# speedup vs baseline: 2.1040x; 1.5546x over previous
import functools

import jax
import jax.numpy as jnp
from jax.experimental import pallas as pl

E = 8
TOPK = 2
D = 768
FF = 2048
R = 64
SCALING = 16.0 / 64.0

TBLK = 256


def _ffn_kernel(x_ref, Wc_ref, bc_ref, wealth_ref,
                Wg_ref, Wu_ref, Wd_ref,
                Agu_ref, BgS_ref, BuS_ref, AdS_ref, Bd2_ref,
                out_ref):
    xb = x_ref[...]

    logits = jnp.dot(xb, Wc_ref[...].T, preferred_element_type=jnp.float32)
    bids = jax.nn.sigmoid(logits + bc_ref[...]) * wealth_ref[...]
    iota = jax.lax.broadcasted_iota(jnp.int32, (TBLK, E), 1)
    m1 = jnp.max(bids, axis=-1, keepdims=True)
    i1 = jnp.min(jnp.where(bids == m1, iota, E), axis=-1, keepdims=True)
    masked = jnp.where(iota == i1, -jnp.inf, bids)
    m2 = jnp.max(masked, axis=-1, keepdims=True)
    i2 = jnp.min(jnp.where(masked == m2, iota, E), axis=-1, keepdims=True)
    w1 = jax.nn.sigmoid(m1 - m2)
    w2 = 1.0 - w1

    blk = jax.lax.broadcasted_iota(jnp.int32, (TBLK, E * R), 1) // R
    mask1 = (blk == i1).astype(jnp.float32)
    mask2 = (blk == i2).astype(jnp.float32)

    G0 = jnp.dot(xb, Wg_ref[...], preferred_element_type=jnp.float32)
    U0 = jnp.dot(xb, Wu_ref[...], preferred_element_type=jnp.float32)

    P = jnp.dot(xb, Agu_ref[...], preferred_element_type=jnp.float32)
    Pg = P[:, :E * R]
    Pu = P[:, E * R:]

    dg1 = jnp.dot(Pg * mask1, BgS_ref[...], preferred_element_type=jnp.float32)
    du1 = jnp.dot(Pu * mask1, BuS_ref[...], preferred_element_type=jnp.float32)
    dg2 = jnp.dot(Pg * mask2, BgS_ref[...], preferred_element_type=jnp.float32)
    du2 = jnp.dot(Pu * mask2, BuS_ref[...], preferred_element_type=jnp.float32)

    h1 = jax.nn.silu(G0 + dg1 * SCALING) * (U0 + du1 * SCALING)
    h2 = jax.nn.silu(G0 + dg2 * SCALING) * (U0 + du2 * SCALING)
    hw1 = w1 * h1
    hw2 = w2 * h2
    H = hw1 + hw2

    pd1 = jnp.dot(hw1, AdS_ref[...], preferred_element_type=jnp.float32) * mask1
    pd2 = jnp.dot(hw2, AdS_ref[...], preferred_element_type=jnp.float32) * mask2
    dlora = jnp.dot(pd1 + pd2, Bd2_ref[...], preferred_element_type=jnp.float32)

    out_ref[...] = (jnp.dot(H, Wd_ref[...], preferred_element_type=jnp.float32)
                    + dlora * SCALING)


@functools.partial(jax.jit, static_argnames=("interpret",))
def _run(x2d, Wc, bc, wealth, W_gate, W_up, W_down,
         A_gate, B_gate, A_up, B_up, A_down, B_down, interpret=False):
    S = x2d.shape[0]
    grid = (S // TBLK,)
    full = lambda *shape: pl.BlockSpec(shape, lambda i: (0,) * len(shape))
    return pl.pallas_call(
        _ffn_kernel,
        grid=grid,
        in_specs=[
            pl.BlockSpec((TBLK, D), lambda i: (i, 0)),
            full(E, D),
            full(1, E),
            full(1, E),
            full(D, FF),
            full(D, FF),
            full(FF, D),
            full(D, 2 * E * R),
            full(E * R, FF),
            full(E * R, FF),
            full(FF, E * R),
            full(E * R, D),
        ],
        out_specs=pl.BlockSpec((TBLK, D), lambda i: (i, 0)),
        out_shape=jax.ShapeDtypeStruct((S, D), jnp.float32),
        interpret=interpret,
    )(x2d, Wc, bc.reshape(1, E), wealth.reshape(1, E), W_gate, W_up, W_down,
      jnp.concatenate([A_gate.transpose(1, 0, 2).reshape(D, E * R),
                       A_up.transpose(1, 0, 2).reshape(D, E * R)],
                      axis=1),
      B_gate.reshape(E * R, FF), B_up.reshape(E * R, FF),
      A_down.transpose(1, 0, 2).reshape(FF, E * R),
      B_down.reshape(E * R, D))


def kernel(x, Wc, bc, wealth, W_gate, W_up, W_down,
           A_gate, B_gate, A_up, B_up, A_down, B_down):
    B, S, _ = x.shape
    out = _run(x.reshape(B * S, D), Wc, bc, wealth, W_gate, W_up, W_down,
               A_gate, B_gate, A_up, B_up, A_down, B_down)
    return out.reshape(B, S, D)


# speedup vs baseline: 2.1436x; 1.0188x over previous
import functools

import jax
import jax.numpy as jnp
from jax.experimental import pallas as pl
from jax.experimental.pallas import tpu as pltpu

E = 8
TOPK = 2
D = 768
FF = 2048
R = 64
SCALING = 16.0 / 64.0

TBLK = 512


def _ffn_kernel(x_ref, Wc_ref, bc_ref, wealth_ref,
                Wg_ref, Wu_ref, Wd_ref,
                Agu_ref, BgS_ref, BuS_ref, AdS_ref, Bd2_ref,
                out_ref):
    xb = x_ref[...]

    logits = jnp.dot(xb, Wc_ref[...].T, preferred_element_type=jnp.float32)
    bids = jax.nn.sigmoid(logits + bc_ref[...]) * wealth_ref[...]
    iota = jax.lax.broadcasted_iota(jnp.int32, (TBLK, E), 1)
    m1 = jnp.max(bids, axis=-1, keepdims=True)
    i1 = jnp.min(jnp.where(bids == m1, iota, E), axis=-1, keepdims=True)
    masked = jnp.where(iota == i1, -jnp.inf, bids)
    m2 = jnp.max(masked, axis=-1, keepdims=True)
    i2 = jnp.min(jnp.where(masked == m2, iota, E), axis=-1, keepdims=True)
    w1 = jax.nn.sigmoid(m1 - m2)
    w2 = 1.0 - w1

    blk = jax.lax.broadcasted_iota(jnp.int32, (TBLK, E * R), 1) // R
    mask1 = (blk == i1).astype(jnp.float32)
    mask2 = (blk == i2).astype(jnp.float32)

    G0 = jnp.dot(xb, Wg_ref[...], preferred_element_type=jnp.float32)
    U0 = jnp.dot(xb, Wu_ref[...], preferred_element_type=jnp.float32)

    P = jnp.dot(xb, Agu_ref[...], preferred_element_type=jnp.float32)
    Pg = P[:, :E * R]
    Pu = P[:, E * R:]

    dg1 = jnp.dot(Pg * mask1, BgS_ref[...], preferred_element_type=jnp.float32)
    du1 = jnp.dot(Pu * mask1, BuS_ref[...], preferred_element_type=jnp.float32)
    dg2 = jnp.dot(Pg * mask2, BgS_ref[...], preferred_element_type=jnp.float32)
    du2 = jnp.dot(Pu * mask2, BuS_ref[...], preferred_element_type=jnp.float32)

    h1 = jax.nn.silu(G0 + dg1 * SCALING) * (U0 + du1 * SCALING)
    h2 = jax.nn.silu(G0 + dg2 * SCALING) * (U0 + du2 * SCALING)
    hw1 = w1 * h1
    hw2 = w2 * h2
    H = hw1 + hw2

    pd1 = jnp.dot(hw1, AdS_ref[...], preferred_element_type=jnp.float32) * mask1
    pd2 = jnp.dot(hw2, AdS_ref[...], preferred_element_type=jnp.float32) * mask2
    dlora = jnp.dot(pd1 + pd2, Bd2_ref[...], preferred_element_type=jnp.float32)

    out_ref[...] = (jnp.dot(H, Wd_ref[...], preferred_element_type=jnp.float32)
                    + dlora * SCALING)


@functools.partial(jax.jit, static_argnames=("interpret",))
def _run(x2d, Wc, bc, wealth, W_gate, W_up, W_down,
         A_gate, B_gate, A_up, B_up, A_down, B_down, interpret=False):
    S = x2d.shape[0]
    grid = (S // TBLK,)
    full = lambda *shape: pl.BlockSpec(shape, lambda i: (0,) * len(shape))
    return pl.pallas_call(
        _ffn_kernel,
        grid=grid,
        in_specs=[
            pl.BlockSpec((TBLK, D), lambda i: (i, 0)),
            full(E, D),
            full(1, E),
            full(1, E),
            full(D, FF),
            full(D, FF),
            full(FF, D),
            full(D, 2 * E * R),
            full(E * R, FF),
            full(E * R, FF),
            full(FF, E * R),
            full(E * R, D),
        ],
        out_specs=pl.BlockSpec((TBLK, D), lambda i: (i, 0)),
        out_shape=jax.ShapeDtypeStruct((S, D), jnp.float32),
        compiler_params=pltpu.CompilerParams(vmem_limit_bytes=63 * 1024 * 1024),
        interpret=interpret,
    )(x2d, Wc, bc.reshape(1, E), wealth.reshape(1, E), W_gate, W_up, W_down,
      jnp.concatenate([A_gate.transpose(1, 0, 2).reshape(D, E * R),
                       A_up.transpose(1, 0, 2).reshape(D, E * R)],
                      axis=1),
      B_gate.reshape(E * R, FF), B_up.reshape(E * R, FF),
      A_down.transpose(1, 0, 2).reshape(FF, E * R),
      B_down.reshape(E * R, D))


def kernel(x, Wc, bc, wealth, W_gate, W_up, W_down,
           A_gate, B_gate, A_up, B_up, A_down, B_down):
    B, S, _ = x.shape
    out = _run(x.reshape(B * S, D), Wc, bc, wealth, W_gate, W_up, W_down,
               A_gate, B_gate, A_up, B_up, A_down, B_down)
    return out.reshape(B, S, D)
